# baseline (speedup 1.0000x reference)
import jax
import jax.numpy as jnp
from jax import lax
from jax.experimental import pallas as pl
from jax.experimental.pallas import tpu as pltpu
from jax.experimental.pallas import tpu_sc as plsc

NUM_CORES = 2
NUM_SUBCORES = 16
NUM_WORKERS = NUM_CORES * NUM_SUBCORES

B = 16384 * 50
D = 32
SUP = 128
B_PER_W = B // NUM_WORKERS
CHUNK = 512
N_CHUNKS = B_PER_W // CHUNK

EXT_ROWS = 2048


def _gather_body(table_hbm, idx_hbm, out_hbm, idx_v, rows_v, sem):
    wid = lax.axis_index("s") * NUM_CORES + lax.axis_index("c")
    base = wid * B_PER_W
    pltpu.sync_copy(idx_hbm.at[pl.ds(base, B_PER_W)], idx_v)

    @pl.loop(0, N_CHUNKS)
    def _(j):
        off = j * CHUNK
        pltpu.async_copy(
            table_hbm.at[idx_v.at[pl.ds(off, CHUNK)]], rows_v, sem
        ).wait()
        pltpu.sync_copy(rows_v, out_hbm.at[pl.ds(base + off, CHUNK)])


def _extract_body(sup_ref, off_ref, out_ref):
    off = off_ref[...]
    acc = jnp.zeros((EXT_ROWS, D), jnp.float32)
    for k in range(4):
        blk = sup_ref[:, k * D:(k + 1) * D]
        acc = jnp.where(off == k, blk, acc)
    out_ref[...] = acc


def kernel(token_ids, embedding_matrix):
    flat_ids = token_ids.reshape(B).astype(jnp.int32)
    sup_ids = jax.lax.shift_right_logical(flat_ids, 2)
    off_ids = jax.lax.bitwise_and(flat_ids, 3).reshape(B, 1)
    table_sup = embedding_matrix.reshape(250000, SUP)

    mesh = plsc.VectorSubcoreMesh(core_axis_name="c", subcore_axis_name="s")
    gather = pl.kernel(
        _gather_body,
        mesh=mesh,
        out_type=jax.ShapeDtypeStruct((B, SUP), jnp.float32),
        scratch_types=[
            pltpu.VMEM((B_PER_W,), jnp.int32),
            pltpu.VMEM((CHUNK, SUP), jnp.float32),
            pltpu.SemaphoreType.DMA,
        ],
    )
    inter = gather(table_sup, sup_ids)

    out = pl.pallas_call(
        _extract_body,
        grid=(B // EXT_ROWS,),
        in_specs=[
            pl.BlockSpec((EXT_ROWS, SUP), lambda i: (i, 0)),
            pl.BlockSpec((EXT_ROWS, 1), lambda i: (i, 0)),
        ],
        out_specs=pl.BlockSpec((EXT_ROWS, D), lambda i: (i, 0)),
        out_shape=jax.ShapeDtypeStruct((B, D), jnp.float32),
    )(inter, off_ids)
    return out.reshape(token_ids.shape + (D,))
